# Initial kernel scaffold; baseline (speedup 1.0000x reference)
import jax
import jax.numpy as jnp
from jax.experimental import pallas as pl


def kernel(feature_idx, feature_value, weight):
    raise NotImplementedError("write your pallas kernel here")


# speedup vs baseline: 1.0868x; 1.0868x over previous
import functools

import jax
import jax.numpy as jnp
from jax import lax
from jax.experimental import pallas as pl
from jax.experimental.pallas import tpu as pltpu
from jax.experimental.pallas import tpu_sc as plsc

NUM_FEATURES = 100000
EMBED_DIM = 64
BATCH = 4096
NUM_FIELDS = 26

B = BATCH * NUM_FIELDS
NC = 2
NS = 16
NW = NC * NS
BPW = B // NW
NCHUNK = 4
CHUNK = BPW // NCHUNK
LANES = 16
VPR = EMBED_DIM // LANES

_mesh = plsc.VectorSubcoreMesh(core_axis_name="c", subcore_axis_name="s")


@functools.partial(
    pl.kernel,
    mesh=_mesh,
    compiler_params=pltpu.CompilerParams(use_tc_tiling_on_sc=False),
    out_type=jax.ShapeDtypeStruct((B, EMBED_DIM), jnp.float32),
    scratch_types=[
        pltpu.VMEM((CHUNK,), jnp.int32),
        pltpu.VMEM((CHUNK,), jnp.float32),
        pltpu.VMEM((CHUNK, EMBED_DIM), jnp.float32),
        pltpu.SemaphoreType.DMA,
    ],
)
def _gather_scale(idx_hbm, val_hbm, w_hbm, out_hbm, idx_v, val_v, rows_v, sem):
    wid = lax.axis_index("s") * NC + lax.axis_index("c")
    base = wid * BPW

    def chunk_body(k, _):
        off = base + k * CHUNK
        pltpu.sync_copy(idx_hbm.at[pl.ds(off, CHUNK)], idx_v)
        pltpu.sync_copy(val_hbm.at[pl.ds(off, CHUNK)], val_v)
        pltpu.async_copy(w_hbm.at[idx_v], rows_v, sem).wait()

        def blk_body(g, _):
            row0 = g * LANES
            vv = val_v[pl.ds(row0, LANES)]
            for l in range(LANES):
                v = jnp.full((LANES,), vv[l], jnp.float32)
                for j in range(VPR):
                    sl = pl.ds(j * LANES, LANES)
                    rows_v[row0 + l, sl] = rows_v[row0 + l, sl] * v
            return 0

        lax.fori_loop(0, CHUNK // LANES, blk_body, 0)
        pltpu.sync_copy(rows_v, out_hbm.at[pl.ds(off, CHUNK)])
        return 0

    lax.fori_loop(0, NCHUNK, chunk_body, 0)


def kernel(feature_idx, feature_value, weight):
    idx = feature_idx.reshape(B)
    val = feature_value.reshape(B).astype(jnp.float32)
    out = _gather_scale(idx, val, weight)
    return out.reshape(BATCH, NUM_FIELDS, EMBED_DIM)
